# Initial kernel scaffold; baseline (speedup 1.0000x reference)
import jax
import jax.numpy as jnp
from jax.experimental import pallas as pl


def kernel(x, edge_index, edge_weight, ln_w, ln_b, W1, b1, W2, b2):
    raise NotImplementedError("write your pallas kernel here")


# speedup vs baseline: 3.0162x; 3.0162x over previous
import functools

import jax
import jax.numpy as jnp
from jax import lax
from jax.experimental import pallas as pl
from jax.experimental.pallas import tpu as pltpu
from jax.experimental.pallas import tpu_sc as plsc

N = 50000
E = 800000
D_IN, D_H, D_OUT = 100, 128, 64

NC, NS, L = 2, 16, 16
ET = E // NS
C = 2000
NCHUNK = ET // C
RING = 4
GB = RING * L


def _dense1(x, ln_w, ln_b, W1, b1):
    BLK = 2000

    def body(x_ref, w_ref, b_ref, lnw_ref, lnb_ref, o_ref):
        xv = jnp.clip(x_ref[...], -1.8, 1.8)
        mu = jnp.mean(xv, axis=1, keepdims=True)
        xc = xv - mu
        var = jnp.mean(xc * xc, axis=1, keepdims=True)
        xn = xc * lax.rsqrt(var + 1e-5) * lnw_ref[...] + lnb_ref[...]
        o_ref[...] = (
            jnp.dot(xn, w_ref[...], preferred_element_type=jnp.float32)
            + b_ref[...]
        )

    return pl.pallas_call(
        body,
        grid=(N // BLK,),
        in_specs=[
            pl.BlockSpec((BLK, D_IN), lambda i: (i, 0)),
            pl.BlockSpec((D_IN, D_H), lambda i: (0, 0)),
            pl.BlockSpec((1, D_H), lambda i: (0, 0)),
            pl.BlockSpec((1, D_IN), lambda i: (0, 0)),
            pl.BlockSpec((1, D_IN), lambda i: (0, 0)),
        ],
        out_specs=pl.BlockSpec((BLK, D_H), lambda i: (i, 0)),
        out_shape=jax.ShapeDtypeStruct((N, D_H), jnp.float32),
    )(x, W1, b1.reshape(1, D_H), ln_w.reshape(1, D_IN), ln_b.reshape(1, D_IN))


def _dense2(a, W2, b2):
    BLK = 2000

    def body(a_ref, w_ref, b_ref, o_ref):
        av = a_ref[...]
        h = jnp.where(av > 0, av, jnp.exp(jnp.minimum(av, 0.0)) - 1.0)
        o_ref[...] = (
            jnp.dot(h, w_ref[...], preferred_element_type=jnp.float32)
            + b_ref[...]
        )

    return pl.pallas_call(
        body,
        grid=(N // BLK,),
        in_specs=[
            pl.BlockSpec((BLK, D_H), lambda i: (i, 0)),
            pl.BlockSpec((D_H, D_OUT), lambda i: (0, 0)),
            pl.BlockSpec((1, D_OUT), lambda i: (0, 0)),
        ],
        out_specs=pl.BlockSpec((BLK, D_OUT), lambda i: (i, 0)),
        out_shape=jax.ShapeDtypeStruct((N, D_OUT), jnp.float32),
    )(a, W2, b2.reshape(1, D_OUT))


def _spmm_sc(D, R, NPASS, h, row, col, w):
    npad = NPASS * NC * R
    stripe = R // NS
    CP = stripe // 7 if stripe % 7 == 0 else L
    assert stripe % L == 0 and R * D * 4 <= 8 * 1024 * 1024 - 4096
    assert npad >= N

    def body(h_ref, row_ref, col_ref, w_ref, out_ref,
             acc, rowc, colc, wc, gbuf, sbuf, gsem, ssem):
        core = lax.axis_index("c")
        s = lax.axis_index("s")
        zf = jnp.zeros((L,), jnp.float32)
        zi = jnp.zeros((L,), jnp.int32)

        def pass_body(p, _):
            base = (p * NC + core) * R
            for r in range(L):
                for k in range(D // L):
                    gbuf[r, pl.ds(k * L, L)] = zf

            def zloop(i, _):
                pltpu.sync_copy(gbuf.at[pl.ds(0, L), :],
                                acc.at[pl.ds(s * stripe + i * L, L), :])
                return 0
            lax.fori_loop(0, stripe // L, zloop, 0)
            plsc.subcore_barrier()

            def chunk_body(cix, _):
                eoff = s * ET + cix * C
                pltpu.sync_copy(row_ref.at[pl.ds(eoff, C)],
                                rowc.at[pl.ds(0, C)])
                pltpu.sync_copy(col_ref.at[pl.ds(eoff, C)],
                                colc.at[pl.ds(0, C)])
                pltpu.sync_copy(w_ref.at[pl.ds(eoff, C)],
                                wc.at[pl.ds(0, C)])

                def scan_body(g, m):
                    r16 = rowc[pl.ds(g * L, L)]
                    c16 = colc[pl.ds(g * L, L)]
                    w16 = wc[pl.ds(g * L, L)]
                    msk = (r16 >= base) & (r16 < base + R)
                    plsc.store_compressed(rowc.at[pl.ds(m, L)], r16 - base,
                                          mask=msk)
                    plsc.store_compressed(colc.at[pl.ds(m, L)], c16,
                                          mask=msk)
                    plsc.store_compressed(wc.at[pl.ds(m, L)], w16, mask=msk)
                    return m + jnp.sum(msk.astype(jnp.int32))

                m = lax.fori_loop(0, C // L, scan_body, jnp.int32(0))

                for k in range(RING):
                    rowc[pl.ds(m + k * L, L)] = zi
                    colc[pl.ds(m + k * L, L)] = zi
                    wc[pl.ds(m + k * L, L)] = zf
                nb = jnp.maximum((m + GB - 1) // GB, 1)

                for slot in range(RING):
                    cvec = colc[pl.ds(slot * L, L)]
                    pltpu.async_copy(h_ref.at[cvec],
                                     gbuf.at[pl.ds(slot * L, L), :],
                                     gsem.at[slot])

                def round_body(t, _):
                    for slot in range(RING):
                        j = t * RING + slot
                        cvec = colc[pl.ds(j * L, L)]
                        pltpu.make_async_copy(
                            h_ref.at[cvec], gbuf.at[pl.ds(slot * L, L), :],
                            gsem.at[slot]).wait()

                        @pl.when(t > 0)
                        def _():
                            jp = (t - 1) * RING + slot
                            lvec = rowc[pl.ds(jp * L, L)]
                            pltpu.make_async_copy(
                                sbuf.at[pl.ds(slot * L, L), :],
                                acc.at[lvec], ssem.at[slot]).wait()

                        wvec = wc[pl.ds(j * L, L)]
                        for r in range(L):
                            wspl = wvec.at[jnp.full((L,), r, jnp.int32)].get(
                                mode="promise_in_bounds")
                            for k in range(D // L):
                                sbuf[slot * L + r, pl.ds(k * L, L)] = (
                                    gbuf[slot * L + r, pl.ds(k * L, L)]
                                    * wspl)

                        lvec = rowc[pl.ds(j * L, L)]
                        pltpu.async_copy(sbuf.at[pl.ds(slot * L, L), :],
                                         acc.at[lvec], ssem.at[slot],
                                         add=True)

                        @pl.when(t + 1 < nb)
                        def _():
                            jn = (t + 1) * RING + slot
                            cvec2 = colc[pl.ds(jn * L, L)]
                            pltpu.async_copy(
                                h_ref.at[cvec2],
                                gbuf.at[pl.ds(slot * L, L), :],
                                gsem.at[slot])
                    return 0

                lax.fori_loop(0, nb, round_body, 0)

                for slot in range(RING):
                    jl = (nb - 1) * RING + slot
                    lvec = rowc[pl.ds(jl * L, L)]
                    pltpu.make_async_copy(sbuf.at[pl.ds(slot * L, L), :],
                                          acc.at[lvec], ssem.at[slot]).wait()
                return 0

            lax.fori_loop(0, NCHUNK, chunk_body, 0)
            plsc.subcore_barrier()

            def cploop(i, _):
                off = s * stripe + i * CP
                pltpu.sync_copy(acc.at[pl.ds(off, CP), :],
                                out_ref.at[pl.ds(base + off, CP), :])
                return 0
            lax.fori_loop(0, stripe // CP, cploop, 0)
            return 0

        lax.fori_loop(0, NPASS, pass_body, 0)

    kern = pl.kernel(
        body,
        out_type=jax.ShapeDtypeStruct((npad, D), jnp.float32),
        mesh=plsc.VectorSubcoreMesh(core_axis_name="c", subcore_axis_name="s"),
        scratch_types=[
            pltpu.VMEM_SHARED((R, D), jnp.float32),
            pltpu.VMEM((C + GB,), jnp.int32),
            pltpu.VMEM((C + GB,), jnp.int32),
            pltpu.VMEM((C + GB,), jnp.float32),
            pltpu.VMEM((GB, D), jnp.float32),
            pltpu.VMEM((GB, D), jnp.float32),
            pltpu.SemaphoreType.DMA((RING,)),
            pltpu.SemaphoreType.DMA((RING,)),
        ],
        compiler_params=pltpu.CompilerParams(needs_layout_passes=False,
                                             use_tc_tiling_on_sc=False),
    )
    return kern(h, row, col, w)[:N]


@jax.jit
def kernel(x, edge_index, edge_weight, ln_w, ln_b, W1, b1, W2, b2):
    row = edge_index[0]
    col = edge_index[1]
    h1 = _dense1(x, ln_w, ln_b, W1, b1)
    a1 = _spmm_sc(D_H, 12544, 2, h1, row, col, edge_weight)
    h2 = _dense2(a1, W2, b2)
    out = _spmm_sc(D_OUT, 25088, 1, h2, row, col, edge_weight)
    return out
